# Initial kernel scaffold; baseline (speedup 1.0000x reference)
import functools

import jax
import jax.numpy as jnp
from jax import lax
from jax.experimental import pallas as pl
from jax.experimental.pallas import tpu as pltpu
from jax.experimental.pallas import tpu_sc as plsc

N, E, D = 10000, 320000, 128
NC, NS = 2, 16
NW = NC * NS
CH = 128
NCHUNKS = E // CH
ITERS = (NCHUNKS + NW - 1) // NW
ROWS_PER_TILE = 632
ROWS_LAST = N - (NS - 1) * ROWS_PER_TILE

_MESH = dict(core_axis_name="c", subcore_axis_name="s")


def _wid():
    return lax.axis_index("s") * NC + lax.axis_index("c")


def _gather_body(row_hbm, col_hbm, x_hbm, src_out, dst_out, idx_v, rows_v, sem):
    wid = _wid()

    def body(i, carry):
        c = wid + i * NW

        @pl.when(c < NCHUNKS)
        def _():
            base = c * CH
            pltpu.sync_copy(row_hbm.at[pl.ds(base, CH)], idx_v)
            pltpu.async_copy(x_hbm.at[idx_v], rows_v, sem).wait()
            pltpu.sync_copy(rows_v, src_out.at[pl.ds(base, CH)])
            pltpu.sync_copy(col_hbm.at[pl.ds(base, CH)], idx_v)
            pltpu.async_copy(x_hbm.at[idx_v], rows_v, sem).wait()
            pltpu.sync_copy(rows_v, dst_out.at[pl.ds(base, CH)])

        return carry

    lax.fori_loop(0, ITERS, body, 0)


_gather = functools.partial(
    pl.kernel,
    out_type=[
        jax.ShapeDtypeStruct((E, D), jnp.float32),
        jax.ShapeDtypeStruct((E, D), jnp.float32),
    ],
    mesh=plsc.VectorSubcoreMesh(**_MESH),
    scratch_types=[
        pltpu.VMEM((CH,), jnp.int32),
        pltpu.VMEM((CH, D), jnp.float32),
        pltpu.SemaphoreType.DMA,
    ],
)(_gather_body)


def _scatter_body(col_hbm, msg_hbm, zrows_hbm, zn_hbm,
                  sum_out, cnt_out, idx_v, msg_v, hist_v, acc_sh, sem):
    cid = lax.axis_index("c")
    sid = lax.axis_index("s")
    wid = sid * NC + cid

    pltpu.sync_copy(zn_hbm, hist_v)
    rbase = sid * ROWS_PER_TILE

    @pl.when(sid < NS - 1)
    def _():
        pltpu.sync_copy(zrows_hbm, acc_sh.at[pl.ds(rbase, ROWS_PER_TILE)])

    @pl.when(sid == NS - 1)
    def _():
        pltpu.sync_copy(zrows_hbm.at[pl.ds(0, ROWS_LAST)],
                        acc_sh.at[pl.ds(rbase, ROWS_LAST)])

    plsc.subcore_barrier()

    ones = jnp.full((16,), 1.0, jnp.float32)

    def body(i, carry):
        c = wid + i * NW

        @pl.when(c < NCHUNKS)
        def _():
            base = c * CH
            pltpu.sync_copy(col_hbm.at[pl.ds(base, CH)], idx_v)
            pltpu.async_copy(msg_hbm.at[pl.ds(base, CH)], msg_v, sem).wait()
            pltpu.sync_copy(msg_v, acc_sh.at[idx_v], add=True)
            for j in range(CH // 16):
                v = idx_v[pl.ds(j * 16, 16)]
                plsc.addupdate_scatter(hist_v, [v], ones)

        return carry

    lax.fori_loop(0, ITERS, body, 0)
    plsc.subcore_barrier()

    pltpu.sync_copy(hist_v, cnt_out.at[wid])

    @pl.when(sid < NS - 1)
    def _():
        pltpu.sync_copy(acc_sh.at[pl.ds(rbase, ROWS_PER_TILE)],
                        sum_out.at[cid, pl.ds(rbase, ROWS_PER_TILE)])

    @pl.when(sid == NS - 1)
    def _():
        pltpu.sync_copy(acc_sh.at[pl.ds(rbase, ROWS_LAST)],
                        sum_out.at[cid, pl.ds(rbase, ROWS_LAST)])


_scatter = functools.partial(
    pl.kernel,
    out_type=[
        jax.ShapeDtypeStruct((NC, N, D), jnp.float32),
        jax.ShapeDtypeStruct((NW, N), jnp.float32),
    ],
    mesh=plsc.VectorSubcoreMesh(**_MESH),
    scratch_types=[
        pltpu.VMEM((CH,), jnp.int32),
        pltpu.VMEM((CH, D), jnp.float32),
        pltpu.VMEM((N,), jnp.float32),
        pltpu.VMEM_SHARED((N, D), jnp.float32),
        pltpu.SemaphoreType.DMA,
    ],
)(_scatter_body)


TE = 2560


def _edge_mlp_body(src, dst, ea, a1, a2, a3, b1e, w2e, b2e,
                   bb1, bb2, b1n, w2n, b2n, ean_out, msg_out):
    f32 = jnp.float32
    eh = jnp.dot(src[...], a1[...], preferred_element_type=f32)
    eh += jnp.dot(dst[...], a2[...], preferred_element_type=f32)
    eh += jnp.dot(ea[...], a3[...], preferred_element_type=f32)
    eh = jnp.maximum(eh + b1e[...], 0.0)
    ean = jnp.dot(eh, w2e[...], preferred_element_type=f32) + b2e[...]
    ean_out[...] = ean
    nh = jnp.dot(src[...], bb1[...], preferred_element_type=f32)
    nh += jnp.dot(ean, bb2[...], preferred_element_type=f32)
    nh = jnp.maximum(nh + b1n[...], 0.0)
    msg_out[...] = jnp.dot(nh, w2n[...], preferred_element_type=f32) + b2n[...]


def _edge_mlp(src, dst, ea, a1, a2, a3, b1e, w2e, b2e, bb1, bb2, b1n, w2n, b2n):
    eblk = pl.BlockSpec((TE, D), lambda i: (i, 0))
    wblk = pl.BlockSpec((D, D), lambda i: (0, 0))
    bblk = pl.BlockSpec((1, D), lambda i: (0, 0))
    return pl.pallas_call(
        _edge_mlp_body,
        grid=(E // TE,),
        in_specs=[eblk, eblk, eblk, wblk, wblk, wblk, bblk, wblk, bblk,
                  wblk, wblk, bblk, wblk, bblk],
        out_specs=[eblk, eblk],
        out_shape=[
            jax.ShapeDtypeStruct((E, D), jnp.float32),
            jax.ShapeDtypeStruct((E, D), jnp.float32),
        ],
    )(src, dst, ea, a1, a2, a3, b1e, w2e, b2e, bb1, bb2, b1n, w2n, b2n)


TN = 2000


def _node_mlp_body(x, s0, s1, cnt, c1, c2, b3n, w4n, b4n, out):
    f32 = jnp.float32
    counts = jnp.sum(cnt[...], axis=0)
    agg = (s0[...] + s1[...]) / jnp.maximum(counts, 1.0)[:, None]
    h2 = jnp.dot(x[...], c1[...], preferred_element_type=f32)
    h2 += jnp.dot(agg, c2[...], preferred_element_type=f32)
    h2 = jnp.maximum(h2 + b3n[...], 0.0)
    out[...] = jnp.dot(h2, w4n[...], preferred_element_type=f32) + b4n[...]


def _node_mlp(x, s0, s1, cnt, c1, c2, b3n, w4n, b4n):
    nblk = pl.BlockSpec((TN, D), lambda i: (i, 0))
    cblk = pl.BlockSpec((NW, TN), lambda i: (0, i))
    wblk = pl.BlockSpec((D, D), lambda i: (0, 0))
    bblk = pl.BlockSpec((1, D), lambda i: (0, 0))
    return pl.pallas_call(
        _node_mlp_body,
        grid=(N // TN,),
        in_specs=[nblk, nblk, nblk, cblk, wblk, wblk, bblk, wblk, bblk],
        out_specs=nblk,
        out_shape=jax.ShapeDtypeStruct((N, D), jnp.float32),
    )(x, s0, s1, cnt, c1, c2, b3n, w4n, b4n)


def kernel(x, edge_index, edge_attr, u, batch,
           W1e, b1e, W2e, b2e, W1n, b1n, W2n, b2n, W3n, b3n, W4n, b4n):
    del u, batch
    row = edge_index[0]
    col = edge_index[1]

    src, dst = _gather(row, col, x)

    a1, a2, a3 = W1e[:D], W1e[D:2 * D], W1e[2 * D:]
    bb1, bb2 = W1n[:D], W1n[D:]
    ean, msg = _edge_mlp(
        src, dst, edge_attr, a1, a2, a3, b1e.reshape(1, D), W2e,
        b2e.reshape(1, D), bb1, bb2, b1n.reshape(1, D), W2n,
        b2n.reshape(1, D))

    zrows = jnp.zeros((ROWS_PER_TILE, D), jnp.float32)
    zn = jnp.zeros((N,), jnp.float32)
    sums, cnts = _scatter(col, msg, zrows, zn)

    c1, c2 = W3n[:D], W3n[D:]
    x_new = _node_mlp(x, sums[0], sums[1], cnts, c1, c2,
                      b3n.reshape(1, D), W4n, b4n.reshape(1, D))
    return (x_new, ean)


# speedup vs baseline: 2.7791x; 2.7791x over previous
import functools

import jax
import jax.numpy as jnp
from jax import lax
from jax.experimental import pallas as pl
from jax.experimental.pallas import tpu as pltpu
from jax.experimental.pallas import tpu_sc as plsc

N, E, D = 10000, 320000, 128
NC, NS = 2, 16
NW = NC * NS
CH = 128
NCHUNKS = E // CH
ITERS = (NCHUNKS + NW - 1) // NW
ITERS2 = (NCHUNKS + NS - 1) // NS
HD = D // NC
ROWS_PER_TILE = 632
ROWS_LAST = N - (NS - 1) * ROWS_PER_TILE
TAIL_MOST = ROWS_PER_TILE - 4 * CH
TAIL_LAST = ROWS_LAST - 4 * CH

_MESH = dict(core_axis_name="c", subcore_axis_name="s")


def _stage_slabs(sid, rbase, copy_fn):
    for k in range(4):
        copy_fn(rbase + k * CH, CH)

    @pl.when(sid < NS - 1)
    def _():
        copy_fn(rbase + 4 * CH, TAIL_MOST)

    @pl.when(sid == NS - 1)
    def _():
        copy_fn(rbase + 4 * CH, TAIL_LAST)


def _gather_body(row_hbm, col_hbm, x_hbm, zc_hbm, ones_hbm,
                 src_out, dst_out, cnt_out,
                 idx_v, rows_v, zc_v, ones_v, cnt_sh, sem):
    cid = lax.axis_index("c")
    sid = lax.axis_index("s")
    wid = sid * NC + cid
    rbase = sid * ROWS_PER_TILE

    pltpu.sync_copy(zc_hbm, zc_v)
    pltpu.sync_copy(ones_hbm, ones_v)

    def zero_chunk(soff, nr):
        pltpu.sync_copy(zc_v.at[pl.ds(0, nr)], cnt_sh.at[pl.ds(soff, nr)])

    _stage_slabs(sid, rbase, zero_chunk)
    plsc.subcore_barrier()

    def body(i, carry):
        c = wid + i * NW

        @pl.when(c < NCHUNKS)
        def _():
            base = c * CH
            pltpu.sync_copy(row_hbm.at[pl.ds(base, CH)], idx_v)
            pltpu.async_copy(x_hbm.at[idx_v], rows_v, sem).wait()
            pltpu.sync_copy(rows_v, src_out.at[pl.ds(base, CH)])
            pltpu.sync_copy(col_hbm.at[pl.ds(base, CH)], idx_v)
            pltpu.async_copy(x_hbm.at[idx_v], rows_v, sem).wait()
            pltpu.sync_copy(rows_v, dst_out.at[pl.ds(base, CH)])
            pltpu.sync_copy(ones_v, cnt_sh.at[idx_v], add=True)

        return carry

    lax.fori_loop(0, ITERS, body, 0)
    plsc.subcore_barrier()

    def write_chunk(soff, nr):
        pltpu.sync_copy(cnt_sh.at[pl.ds(soff, nr)], zc_v.at[pl.ds(0, nr)])
        pltpu.sync_copy(zc_v.at[pl.ds(0, nr)],
                        cnt_out.at[pl.ds(cid * N + soff, nr)])

    _stage_slabs(sid, rbase, write_chunk)


_gather = functools.partial(
    pl.kernel,
    out_type=[
        jax.ShapeDtypeStruct((E, D), jnp.float32),
        jax.ShapeDtypeStruct((E, D), jnp.float32),
        jax.ShapeDtypeStruct((NC * N, 16), jnp.float32),
    ],
    mesh=plsc.VectorSubcoreMesh(**_MESH),
    scratch_types=[
        pltpu.VMEM((CH,), jnp.int32),
        pltpu.VMEM((CH, D), jnp.float32),
        pltpu.VMEM((CH, 16), jnp.float32),
        pltpu.VMEM((CH, 16), jnp.float32),
        pltpu.VMEM_SHARED((N, 16), jnp.float32),
        pltpu.SemaphoreType.DMA,
    ],
)(_gather_body)


def _scatter_body(col_hbm, mlo_hbm, mhi_hbm, zrows_hbm,
                  slo_out, shi_out, idx_v, msg_v, buf_v, acc_sh, sem):
    cid = lax.axis_index("c")
    sid = lax.axis_index("s")
    rbase = sid * ROWS_PER_TILE

    pltpu.sync_copy(zrows_hbm, buf_v)

    def zero_chunk(soff, nr):
        pltpu.sync_copy(buf_v.at[pl.ds(0, nr)], acc_sh.at[pl.ds(soff, nr)])

    _stage_slabs(sid, rbase, zero_chunk)
    plsc.subcore_barrier()

    def body(i, carry):
        c = sid + i * NS

        @pl.when(c < NCHUNKS)
        def _():
            base = c * CH
            pltpu.sync_copy(col_hbm.at[pl.ds(base, CH)], idx_v)

            @pl.when(cid == 0)
            def _():
                pltpu.async_copy(mlo_hbm.at[pl.ds(base, CH)], msg_v, sem).wait()

            @pl.when(cid == 1)
            def _():
                pltpu.async_copy(mhi_hbm.at[pl.ds(base, CH)], msg_v, sem).wait()

            pltpu.sync_copy(msg_v, acc_sh.at[idx_v], add=True)

        return carry

    lax.fori_loop(0, ITERS2, body, 0)
    plsc.subcore_barrier()

    def write_chunk(soff, nr):
        pltpu.sync_copy(acc_sh.at[pl.ds(soff, nr)], buf_v.at[pl.ds(0, nr)])

        @pl.when(cid == 0)
        def _():
            pltpu.sync_copy(buf_v.at[pl.ds(0, nr)], slo_out.at[pl.ds(soff, nr)])

        @pl.when(cid == 1)
        def _():
            pltpu.sync_copy(buf_v.at[pl.ds(0, nr)], shi_out.at[pl.ds(soff, nr)])

    _stage_slabs(sid, rbase, write_chunk)


_scatter = functools.partial(
    pl.kernel,
    out_type=[
        jax.ShapeDtypeStruct((N, HD), jnp.float32),
        jax.ShapeDtypeStruct((N, HD), jnp.float32),
    ],
    mesh=plsc.VectorSubcoreMesh(**_MESH),
    scratch_types=[
        pltpu.VMEM((CH,), jnp.int32),
        pltpu.VMEM((CH, HD), jnp.float32),
        pltpu.VMEM((CH, HD), jnp.float32),
        pltpu.VMEM_SHARED((N, HD), jnp.float32),
        pltpu.SemaphoreType.DMA,
    ],
)(_scatter_body)


TE = 2560


def _edge_mlp_body(src, dst, ea, a1, a2, a3, b1e, w2e, b2e,
                   bb1, bb2, b1n, w2n, b2n, ean_out, mlo_out, mhi_out):
    f32 = jnp.float32
    eh = jnp.dot(src[...], a1[...], preferred_element_type=f32)
    eh += jnp.dot(dst[...], a2[...], preferred_element_type=f32)
    eh += jnp.dot(ea[...], a3[...], preferred_element_type=f32)
    eh = jnp.maximum(eh + b1e[...], 0.0)
    ean = jnp.dot(eh, w2e[...], preferred_element_type=f32) + b2e[...]
    ean_out[...] = ean
    nh = jnp.dot(src[...], bb1[...], preferred_element_type=f32)
    nh += jnp.dot(ean, bb2[...], preferred_element_type=f32)
    nh = jnp.maximum(nh + b1n[...], 0.0)
    msg = jnp.dot(nh, w2n[...], preferred_element_type=f32) + b2n[...]
    mlo_out[...] = msg[:, :HD]
    mhi_out[...] = msg[:, HD:]


def _edge_mlp(src, dst, ea, a1, a2, a3, b1e, w2e, b2e, bb1, bb2, b1n, w2n, b2n):
    eblk = pl.BlockSpec((TE, D), lambda i: (i, 0))
    hblk = pl.BlockSpec((TE, HD), lambda i: (i, 0))
    wblk = pl.BlockSpec((D, D), lambda i: (0, 0))
    bblk = pl.BlockSpec((1, D), lambda i: (0, 0))
    return pl.pallas_call(
        _edge_mlp_body,
        grid=(E // TE,),
        in_specs=[eblk, eblk, eblk, wblk, wblk, wblk, bblk, wblk, bblk,
                  wblk, wblk, bblk, wblk, bblk],
        out_specs=[eblk, hblk, hblk],
        out_shape=[
            jax.ShapeDtypeStruct((E, D), jnp.float32),
            jax.ShapeDtypeStruct((E, HD), jnp.float32),
            jax.ShapeDtypeStruct((E, HD), jnp.float32),
        ],
    )(src, dst, ea, a1, a2, a3, b1e, w2e, b2e, bb1, bb2, b1n, w2n, b2n)


TN = 2000


def _node_mlp_body(x, slo, shi, cnt0, cnt1, c1, c2a, c2b, b3n, w4n, b4n, out):
    f32 = jnp.float32
    counts = cnt0[...][:, :1] + cnt1[...][:, :1]
    inv = 1.0 / jnp.maximum(counts, 1.0)
    h2 = jnp.dot(x[...], c1[...], preferred_element_type=f32)
    h2 += jnp.dot(slo[...] * inv, c2a[...], preferred_element_type=f32)
    h2 += jnp.dot(shi[...] * inv, c2b[...], preferred_element_type=f32)
    h2 = jnp.maximum(h2 + b3n[...], 0.0)
    out[...] = jnp.dot(h2, w4n[...], preferred_element_type=f32) + b4n[...]


def _node_mlp(x, slo, shi, cnt0, cnt1, c1, c2a, c2b, b3n, w4n, b4n):
    nblk = pl.BlockSpec((TN, D), lambda i: (i, 0))
    hblk = pl.BlockSpec((TN, HD), lambda i: (i, 0))
    cblk = pl.BlockSpec((TN, 16), lambda i: (i, 0))
    wblk = pl.BlockSpec((D, D), lambda i: (0, 0))
    hwblk = pl.BlockSpec((HD, D), lambda i: (0, 0))
    bblk = pl.BlockSpec((1, D), lambda i: (0, 0))
    return pl.pallas_call(
        _node_mlp_body,
        grid=(N // TN,),
        in_specs=[nblk, hblk, hblk, cblk, cblk, wblk, hwblk, hwblk, bblk,
                  wblk, bblk],
        out_specs=nblk,
        out_shape=jax.ShapeDtypeStruct((N, D), jnp.float32),
    )(x, slo, shi, cnt0, cnt1, c1, c2a, c2b, b3n, w4n, b4n)


def kernel(x, edge_index, edge_attr, u, batch,
           W1e, b1e, W2e, b2e, W1n, b1n, W2n, b2n, W3n, b3n, W4n, b4n):
    del u, batch
    row = edge_index[0]
    col = edge_index[1]

    zc = jnp.zeros((CH, 16), jnp.float32)
    onesr = jnp.ones((CH, 16), jnp.float32)
    src, dst, cnt = _gather(row, col, x, zc, onesr)

    a1, a2, a3 = W1e[:D], W1e[D:2 * D], W1e[2 * D:]
    bb1, bb2 = W1n[:D], W1n[D:]
    ean, mlo, mhi = _edge_mlp(
        src, dst, edge_attr, a1, a2, a3, b1e.reshape(1, D), W2e,
        b2e.reshape(1, D), bb1, bb2, b1n.reshape(1, D), W2n,
        b2n.reshape(1, D))

    zrows = jnp.zeros((CH, HD), jnp.float32)
    slo, shi = _scatter(col, mlo, mhi, zrows)

    c1, c2a, c2b = W3n[:D], W3n[D:D + HD], W3n[D + HD:]
    x_new = _node_mlp(x, slo, shi, cnt[:N], cnt[N:], c1, c2a, c2b,
                      b3n.reshape(1, D), W4n, b4n.reshape(1, D))
    return (x_new, ean)


# speedup vs baseline: 2.7829x; 1.0014x over previous
import functools

import jax
import jax.numpy as jnp
from jax import lax
from jax.experimental import pallas as pl
from jax.experimental.pallas import tpu as pltpu
from jax.experimental.pallas import tpu_sc as plsc

N, E, D = 10000, 320000, 128
NC, NS = 2, 16
NW = NC * NS
CH = 128
NCHUNKS = E // CH
ITERS = (NCHUNKS + NW - 1) // NW
ITERS2 = (NCHUNKS + NS - 1) // NS
HD = D // NC
ROWS_PER_TILE = 632
ROWS_LAST = N - (NS - 1) * ROWS_PER_TILE
TAIL_MOST = ROWS_PER_TILE - 4 * CH
TAIL_LAST = ROWS_LAST - 4 * CH

_MESH = dict(core_axis_name="c", subcore_axis_name="s")


def _stage_slabs(sid, rbase, copy_fn):
    for k in range(4):
        copy_fn(rbase + k * CH, CH)

    @pl.when(sid < NS - 1)
    def _():
        copy_fn(rbase + 4 * CH, TAIL_MOST)

    @pl.when(sid == NS - 1)
    def _():
        copy_fn(rbase + 4 * CH, TAIL_LAST)


def _gather_body(row_hbm, col_hbm, x_hbm, zc_hbm, ones_hbm,
                 src_out, dst_out, cnt_out,
                 idx_v, rows_v, zc_v, ones_v, cnt_sh, sem):
    cid = lax.axis_index("c")
    sid = lax.axis_index("s")
    wid = sid * NC + cid
    rbase = sid * ROWS_PER_TILE

    pltpu.sync_copy(zc_hbm, zc_v)
    pltpu.sync_copy(ones_hbm, ones_v)

    def zero_chunk(soff, nr):
        pltpu.sync_copy(zc_v.at[pl.ds(0, nr)], cnt_sh.at[pl.ds(soff, nr)])

    _stage_slabs(sid, rbase, zero_chunk)
    plsc.subcore_barrier()

    def body(i, carry):
        c = wid + i * NW

        @pl.when(c < NCHUNKS)
        def _():
            base = c * CH
            pltpu.sync_copy(row_hbm.at[pl.ds(base, CH)], idx_v)
            pltpu.async_copy(x_hbm.at[idx_v], rows_v, sem).wait()
            pltpu.sync_copy(rows_v, src_out.at[pl.ds(base, CH)])
            pltpu.sync_copy(col_hbm.at[pl.ds(base, CH)], idx_v)
            pltpu.async_copy(x_hbm.at[idx_v], rows_v, sem).wait()
            pltpu.sync_copy(rows_v, dst_out.at[pl.ds(base, CH)])
            pltpu.sync_copy(ones_v, cnt_sh.at[idx_v], add=True)

        return carry

    lax.fori_loop(0, ITERS, body, 0)
    plsc.subcore_barrier()

    def write_chunk(soff, nr):
        pltpu.sync_copy(cnt_sh.at[pl.ds(soff, nr)], zc_v.at[pl.ds(0, nr)])
        pltpu.sync_copy(zc_v.at[pl.ds(0, nr)],
                        cnt_out.at[pl.ds(cid * N + soff, nr)])

    _stage_slabs(sid, rbase, write_chunk)


_gather = functools.partial(
    pl.kernel,
    out_type=[
        jax.ShapeDtypeStruct((E, D), jnp.float32),
        jax.ShapeDtypeStruct((E, D), jnp.float32),
        jax.ShapeDtypeStruct((NC * N, 16), jnp.float32),
    ],
    mesh=plsc.VectorSubcoreMesh(**_MESH),
    scratch_types=[
        pltpu.VMEM((CH,), jnp.int32),
        pltpu.VMEM((CH, D), jnp.float32),
        pltpu.VMEM((CH, 16), jnp.float32),
        pltpu.VMEM((CH, 16), jnp.float32),
        pltpu.VMEM_SHARED((N, 16), jnp.float32),
        pltpu.SemaphoreType.DMA,
    ],
)(_gather_body)


def _scatter_body(col_hbm, mlo_hbm, mhi_hbm, zrows_hbm,
                  slo_out, shi_out, idx_v, msg_v, buf_v, acc_sh, sem):
    cid = lax.axis_index("c")
    sid = lax.axis_index("s")
    rbase = sid * ROWS_PER_TILE

    pltpu.sync_copy(zrows_hbm, buf_v)

    def zero_chunk(soff, nr):
        pltpu.sync_copy(buf_v.at[pl.ds(0, nr)], acc_sh.at[pl.ds(soff, nr)])

    _stage_slabs(sid, rbase, zero_chunk)
    plsc.subcore_barrier()

    def body(i, carry):
        c = sid + i * NS

        @pl.when(c < NCHUNKS)
        def _():
            base = c * CH
            pltpu.sync_copy(col_hbm.at[pl.ds(base, CH)], idx_v)

            @pl.when(cid == 0)
            def _():
                pltpu.async_copy(mlo_hbm.at[pl.ds(base, CH)], msg_v, sem).wait()

            @pl.when(cid == 1)
            def _():
                pltpu.async_copy(mhi_hbm.at[pl.ds(base, CH)], msg_v, sem).wait()

            pltpu.sync_copy(msg_v, acc_sh.at[idx_v], add=True)

        return carry

    lax.fori_loop(0, ITERS2, body, 0)
    plsc.subcore_barrier()

    def write_chunk(soff, nr):
        pltpu.sync_copy(acc_sh.at[pl.ds(soff, nr)], buf_v.at[pl.ds(0, nr)])

        @pl.when(cid == 0)
        def _():
            pltpu.sync_copy(buf_v.at[pl.ds(0, nr)], slo_out.at[pl.ds(soff, nr)])

        @pl.when(cid == 1)
        def _():
            pltpu.sync_copy(buf_v.at[pl.ds(0, nr)], shi_out.at[pl.ds(soff, nr)])

    _stage_slabs(sid, rbase, write_chunk)


_scatter = functools.partial(
    pl.kernel,
    out_type=[
        jax.ShapeDtypeStruct((N, HD), jnp.float32),
        jax.ShapeDtypeStruct((N, HD), jnp.float32),
    ],
    mesh=plsc.VectorSubcoreMesh(**_MESH),
    scratch_types=[
        pltpu.VMEM((CH,), jnp.int32),
        pltpu.VMEM((CH, HD), jnp.float32),
        pltpu.VMEM((CH, HD), jnp.float32),
        pltpu.VMEM_SHARED((N, HD), jnp.float32),
        pltpu.SemaphoreType.DMA,
    ],
)(_scatter_body)


TE = 2560


def _edge_mlp_body(src, dst, ea, a1, a2, a3, b1e, w2e, b2e,
                   bb1, bb2, b1n, w2n, b2n, ean_out, mlo_out, mhi_out):
    f32, bf16 = jnp.float32, jnp.bfloat16
    sb, db = src[...].astype(bf16), dst[...].astype(bf16)
    eh = jnp.dot(sb, a1[...].astype(bf16), preferred_element_type=f32)
    eh += jnp.dot(db, a2[...].astype(bf16), preferred_element_type=f32)
    eh += jnp.dot(ea[...].astype(bf16), a3[...].astype(bf16),
                  preferred_element_type=f32)
    eh = jnp.maximum(eh + b1e[...], 0.0)
    ean = jnp.dot(eh.astype(bf16), w2e[...].astype(bf16),
                  preferred_element_type=f32) + b2e[...]
    ean_out[...] = ean
    nh = jnp.dot(sb, bb1[...].astype(bf16), preferred_element_type=f32)
    nh += jnp.dot(ean.astype(bf16), bb2[...].astype(bf16),
                  preferred_element_type=f32)
    nh = jnp.maximum(nh + b1n[...], 0.0)
    msg = jnp.dot(nh.astype(bf16), w2n[...].astype(bf16),
                  preferred_element_type=f32) + b2n[...]
    mlo_out[...] = msg[:, :HD]
    mhi_out[...] = msg[:, HD:]


def _edge_mlp(src, dst, ea, a1, a2, a3, b1e, w2e, b2e, bb1, bb2, b1n, w2n, b2n):
    eblk = pl.BlockSpec((TE, D), lambda i: (i, 0))
    hblk = pl.BlockSpec((TE, HD), lambda i: (i, 0))
    wblk = pl.BlockSpec((D, D), lambda i: (0, 0))
    bblk = pl.BlockSpec((1, D), lambda i: (0, 0))
    return pl.pallas_call(
        _edge_mlp_body,
        grid=(E // TE,),
        in_specs=[eblk, eblk, eblk, wblk, wblk, wblk, bblk, wblk, bblk,
                  wblk, wblk, bblk, wblk, bblk],
        out_specs=[eblk, hblk, hblk],
        out_shape=[
            jax.ShapeDtypeStruct((E, D), jnp.float32),
            jax.ShapeDtypeStruct((E, HD), jnp.float32),
            jax.ShapeDtypeStruct((E, HD), jnp.float32),
        ],
    )(src, dst, ea, a1, a2, a3, b1e, w2e, b2e, bb1, bb2, b1n, w2n, b2n)


TN = 2000


def _node_mlp_body(x, slo, shi, cnt0, cnt1, c1, c2a, c2b, b3n, w4n, b4n, out):
    f32 = jnp.float32
    counts = cnt0[...][:, :1] + cnt1[...][:, :1]
    inv = 1.0 / jnp.maximum(counts, 1.0)
    h2 = jnp.dot(x[...], c1[...], preferred_element_type=f32)
    h2 += jnp.dot(slo[...] * inv, c2a[...], preferred_element_type=f32)
    h2 += jnp.dot(shi[...] * inv, c2b[...], preferred_element_type=f32)
    h2 = jnp.maximum(h2 + b3n[...], 0.0)
    out[...] = jnp.dot(h2, w4n[...], preferred_element_type=f32) + b4n[...]


def _node_mlp(x, slo, shi, cnt0, cnt1, c1, c2a, c2b, b3n, w4n, b4n):
    nblk = pl.BlockSpec((TN, D), lambda i: (i, 0))
    hblk = pl.BlockSpec((TN, HD), lambda i: (i, 0))
    cblk = pl.BlockSpec((TN, 16), lambda i: (i, 0))
    wblk = pl.BlockSpec((D, D), lambda i: (0, 0))
    hwblk = pl.BlockSpec((HD, D), lambda i: (0, 0))
    bblk = pl.BlockSpec((1, D), lambda i: (0, 0))
    return pl.pallas_call(
        _node_mlp_body,
        grid=(N // TN,),
        in_specs=[nblk, hblk, hblk, cblk, cblk, wblk, hwblk, hwblk, bblk,
                  wblk, bblk],
        out_specs=nblk,
        out_shape=jax.ShapeDtypeStruct((N, D), jnp.float32),
    )(x, slo, shi, cnt0, cnt1, c1, c2a, c2b, b3n, w4n, b4n)


def kernel(x, edge_index, edge_attr, u, batch,
           W1e, b1e, W2e, b2e, W1n, b1n, W2n, b2n, W3n, b3n, W4n, b4n):
    del u, batch
    row = edge_index[0]
    col = edge_index[1]

    zc = jnp.zeros((CH, 16), jnp.float32)
    onesr = jnp.ones((CH, 16), jnp.float32)
    src, dst, cnt = _gather(row, col, x, zc, onesr)

    a1, a2, a3 = W1e[:D], W1e[D:2 * D], W1e[2 * D:]
    bb1, bb2 = W1n[:D], W1n[D:]
    ean, mlo, mhi = _edge_mlp(
        src, dst, edge_attr, a1, a2, a3, b1e.reshape(1, D), W2e,
        b2e.reshape(1, D), bb1, bb2, b1n.reshape(1, D), W2n,
        b2n.reshape(1, D))

    zrows = jnp.zeros((CH, HD), jnp.float32)
    slo, shi = _scatter(col, mlo, mhi, zrows)

    c1, c2a, c2b = W3n[:D], W3n[D:D + HD], W3n[D + HD:]
    x_new = _node_mlp(x, slo, shi, cnt[:N], cnt[N:], c1, c2a, c2b,
                      b3n.reshape(1, D), W4n, b4n.reshape(1, D))
    return (x_new, ean)


# speedup vs baseline: 2.7860x; 1.0011x over previous
import functools

import jax
import jax.numpy as jnp
from jax import lax
from jax.experimental import pallas as pl
from jax.experimental.pallas import tpu as pltpu
from jax.experimental.pallas import tpu_sc as plsc

N, E, D = 10000, 320000, 128
NC, NS = 2, 16
NW = NC * NS
CH = 128
NCHUNKS = E // CH
ITERS = (NCHUNKS + NW - 1) // NW
ITERS2 = (NCHUNKS + NS - 1) // NS
HD = D // NC
ROWS_PER_TILE = 632
ROWS_LAST = N - (NS - 1) * ROWS_PER_TILE
TAIL_MOST = ROWS_PER_TILE - 4 * CH
TAIL_LAST = ROWS_LAST - 4 * CH

_MESH = dict(core_axis_name="c", subcore_axis_name="s")


def _stage_slabs(sid, rbase, copy_fn):
    for k in range(4):
        copy_fn(rbase + k * CH, CH)

    @pl.when(sid < NS - 1)
    def _():
        copy_fn(rbase + 4 * CH, TAIL_MOST)

    @pl.when(sid == NS - 1)
    def _():
        copy_fn(rbase + 4 * CH, TAIL_LAST)


def _gather_body(row_hbm, col_hbm, x_hbm, zc_hbm, ones_hbm,
                 src_out, dst_out, cnt_out,
                 idx_v, rows_v, zc_v, ones_v, cnt_sh, sem):
    cid = lax.axis_index("c")
    sid = lax.axis_index("s")
    wid = sid * NC + cid
    rbase = sid * ROWS_PER_TILE

    pltpu.sync_copy(zc_hbm, zc_v)
    pltpu.sync_copy(ones_hbm, ones_v)

    def zero_chunk(soff, nr):
        pltpu.sync_copy(zc_v.at[pl.ds(0, nr)], cnt_sh.at[pl.ds(soff, nr)])

    _stage_slabs(sid, rbase, zero_chunk)
    plsc.subcore_barrier()

    def body(i, carry):
        c = wid + i * NW

        @pl.when(c < NCHUNKS)
        def _():
            base = c * CH
            pltpu.sync_copy(row_hbm.at[pl.ds(base, CH)], idx_v)
            pltpu.async_copy(x_hbm.at[idx_v], rows_v, sem).wait()
            pltpu.sync_copy(rows_v, src_out.at[pl.ds(base, CH)])
            pltpu.sync_copy(col_hbm.at[pl.ds(base, CH)], idx_v)
            pltpu.async_copy(x_hbm.at[idx_v], rows_v, sem).wait()
            pltpu.sync_copy(rows_v, dst_out.at[pl.ds(base, CH)])
            pltpu.sync_copy(ones_v, cnt_sh.at[idx_v], add=True)

        return carry

    lax.fori_loop(0, ITERS, body, 0)
    plsc.subcore_barrier()

    def write_chunk(soff, nr):
        pltpu.sync_copy(cnt_sh.at[pl.ds(soff, nr)], zc_v.at[pl.ds(0, nr)])
        pltpu.sync_copy(zc_v.at[pl.ds(0, nr)],
                        cnt_out.at[pl.ds(cid * N + soff, nr)])

    _stage_slabs(sid, rbase, write_chunk)


_gather = functools.partial(
    pl.kernel,
    out_type=[
        jax.ShapeDtypeStruct((E, D), jnp.float32),
        jax.ShapeDtypeStruct((E, D), jnp.float32),
        jax.ShapeDtypeStruct((NC * N, 16), jnp.float32),
    ],
    mesh=plsc.VectorSubcoreMesh(**_MESH),
    scratch_types=[
        pltpu.VMEM((CH,), jnp.int32),
        pltpu.VMEM((CH, D), jnp.float32),
        pltpu.VMEM((CH, 16), jnp.float32),
        pltpu.VMEM((CH, 16), jnp.float32),
        pltpu.VMEM_SHARED((N, 16), jnp.float32),
        pltpu.SemaphoreType.DMA,
    ],
)(_gather_body)


def _scatter_body(col_hbm, mlo_hbm, mhi_hbm, zrows_hbm,
                  slo_out, shi_out, idx_v, msg_v, buf_v, acc_sh, sem):
    cid = lax.axis_index("c")
    sid = lax.axis_index("s")
    rbase = sid * ROWS_PER_TILE

    pltpu.sync_copy(zrows_hbm, buf_v)

    def zero_chunk(soff, nr):
        pltpu.sync_copy(buf_v.at[pl.ds(0, nr)], acc_sh.at[pl.ds(soff, nr)])

    _stage_slabs(sid, rbase, zero_chunk)
    plsc.subcore_barrier()

    def body(i, carry):
        c = sid + i * NS

        @pl.when(c < NCHUNKS)
        def _():
            base = c * CH
            pltpu.sync_copy(col_hbm.at[pl.ds(base, CH)], idx_v)

            @pl.when(cid == 0)
            def _():
                pltpu.async_copy(mlo_hbm.at[pl.ds(base, CH)], msg_v, sem).wait()

            @pl.when(cid == 1)
            def _():
                pltpu.async_copy(mhi_hbm.at[pl.ds(base, CH)], msg_v, sem).wait()

            pltpu.sync_copy(msg_v, acc_sh.at[idx_v], add=True)

        return carry

    lax.fori_loop(0, ITERS2, body, 0)
    plsc.subcore_barrier()

    def write_chunk(soff, nr):
        pltpu.sync_copy(acc_sh.at[pl.ds(soff, nr)], buf_v.at[pl.ds(0, nr)])

        @pl.when(cid == 0)
        def _():
            pltpu.sync_copy(buf_v.at[pl.ds(0, nr)], slo_out.at[pl.ds(soff, nr)])

        @pl.when(cid == 1)
        def _():
            pltpu.sync_copy(buf_v.at[pl.ds(0, nr)], shi_out.at[pl.ds(soff, nr)])

    _stage_slabs(sid, rbase, write_chunk)


_scatter = functools.partial(
    pl.kernel,
    out_type=[
        jax.ShapeDtypeStruct((N, HD), jnp.float32),
        jax.ShapeDtypeStruct((N, HD), jnp.float32),
    ],
    mesh=plsc.VectorSubcoreMesh(**_MESH),
    scratch_types=[
        pltpu.VMEM((CH,), jnp.int32),
        pltpu.VMEM((CH, HD), jnp.float32),
        pltpu.VMEM((CH, HD), jnp.float32),
        pltpu.VMEM_SHARED((N, HD), jnp.float32),
        pltpu.SemaphoreType.DMA,
    ],
)(_scatter_body)


TE = 2560


def _edge_mlp_body(src, dst, ea, a1, a2, a3, b1e, w2e, b2e,
                   bb1, bb2, b1n, w2n, b2n, ean_out, mlo_out, mhi_out):
    f32 = jnp.float32
    eh = jnp.dot(src[...], a1[...], preferred_element_type=f32)
    eh += jnp.dot(dst[...], a2[...], preferred_element_type=f32)
    eh += jnp.dot(ea[...], a3[...], preferred_element_type=f32)
    eh = jnp.maximum(eh + b1e[...], 0.0)
    ean = jnp.dot(eh, w2e[...], preferred_element_type=f32) + b2e[...]
    ean_out[...] = ean
    nh = jnp.dot(src[...], bb1[...], preferred_element_type=f32)
    nh += jnp.dot(ean, bb2[...], preferred_element_type=f32)
    nh = jnp.maximum(nh + b1n[...], 0.0)
    msg = jnp.dot(nh, w2n[...], preferred_element_type=f32) + b2n[...]
    mlo_out[...] = msg[:, :HD]
    mhi_out[...] = msg[:, HD:]


def _edge_mlp(src, dst, ea, a1, a2, a3, b1e, w2e, b2e, bb1, bb2, b1n, w2n, b2n):
    eblk = pl.BlockSpec((TE, D), lambda i: (i, 0))
    hblk = pl.BlockSpec((TE, HD), lambda i: (i, 0))
    wblk = pl.BlockSpec((D, D), lambda i: (0, 0))
    bblk = pl.BlockSpec((1, D), lambda i: (0, 0))
    return pl.pallas_call(
        _edge_mlp_body,
        grid=(E // TE,),
        in_specs=[eblk, eblk, eblk, wblk, wblk, wblk, bblk, wblk, bblk,
                  wblk, wblk, bblk, wblk, bblk],
        out_specs=[eblk, hblk, hblk],
        out_shape=[
            jax.ShapeDtypeStruct((E, D), jnp.float32),
            jax.ShapeDtypeStruct((E, HD), jnp.float32),
            jax.ShapeDtypeStruct((E, HD), jnp.float32),
        ],
    )(src, dst, ea, a1, a2, a3, b1e, w2e, b2e, bb1, bb2, b1n, w2n, b2n)


TN = 2000


def _node_mlp_body(x, slo, shi, cnt0, cnt1, c1, c2a, c2b, b3n, w4n, b4n, out):
    f32 = jnp.float32
    counts = cnt0[...][:, :1] + cnt1[...][:, :1]
    inv = 1.0 / jnp.maximum(counts, 1.0)
    h2 = jnp.dot(x[...], c1[...], preferred_element_type=f32)
    h2 += jnp.dot(slo[...] * inv, c2a[...], preferred_element_type=f32)
    h2 += jnp.dot(shi[...] * inv, c2b[...], preferred_element_type=f32)
    h2 = jnp.maximum(h2 + b3n[...], 0.0)
    out[...] = jnp.dot(h2, w4n[...], preferred_element_type=f32) + b4n[...]


def _node_mlp(x, slo, shi, cnt0, cnt1, c1, c2a, c2b, b3n, w4n, b4n):
    nblk = pl.BlockSpec((TN, D), lambda i: (i, 0))
    hblk = pl.BlockSpec((TN, HD), lambda i: (i, 0))
    cblk = pl.BlockSpec((TN, 16), lambda i: (i, 0))
    wblk = pl.BlockSpec((D, D), lambda i: (0, 0))
    hwblk = pl.BlockSpec((HD, D), lambda i: (0, 0))
    bblk = pl.BlockSpec((1, D), lambda i: (0, 0))
    return pl.pallas_call(
        _node_mlp_body,
        grid=(N // TN,),
        in_specs=[nblk, hblk, hblk, cblk, cblk, wblk, hwblk, hwblk, bblk,
                  wblk, bblk],
        out_specs=nblk,
        out_shape=jax.ShapeDtypeStruct((N, D), jnp.float32),
    )(x, slo, shi, cnt0, cnt1, c1, c2a, c2b, b3n, w4n, b4n)


def kernel(x, edge_index, edge_attr, u, batch,
           W1e, b1e, W2e, b2e, W1n, b1n, W2n, b2n, W3n, b3n, W4n, b4n):
    del u, batch
    row = edge_index[0]
    col = edge_index[1]

    zc = jnp.zeros((CH, 16), jnp.float32)
    onesr = jnp.ones((CH, 16), jnp.float32)
    src, dst, cnt = _gather(row, col, x, zc, onesr)

    a1, a2, a3 = W1e[:D], W1e[D:2 * D], W1e[2 * D:]
    bb1, bb2 = W1n[:D], W1n[D:]
    ean, mlo, mhi = _edge_mlp(
        src, dst, edge_attr, a1, a2, a3, b1e.reshape(1, D), W2e,
        b2e.reshape(1, D), bb1, bb2, b1n.reshape(1, D), W2n,
        b2n.reshape(1, D))

    zrows = jnp.zeros((CH, HD), jnp.float32)
    slo, shi = _scatter(col, mlo, mhi, zrows)

    c1, c2a, c2b = W3n[:D], W3n[D:D + HD], W3n[D + HD:]
    x_new = _node_mlp(x, slo, shi, cnt[:N], cnt[N:], c1, c2a, c2b,
                      b3n.reshape(1, D), W4n, b4n.reshape(1, D))
    return (x_new, ean)
